# baseline (speedup 1.0000x reference)
import jax
import jax.numpy as jnp
from jax.experimental import pallas as pl
from jax.experimental.pallas import tpu as pltpu

N_ROWS = 16384
N_COLS = 16320


BLK_C = 2048
GRID = N_ROWS // BLK_C


def _gather_row_body(idx_ref, block_ref, out_ref):
    s = idx_ref[0] % 8
    out_ref[...] = block_ref[pl.ds(s, 1), :]


_gather_row = pl.pallas_call(
    _gather_row_body,
    grid_spec=pltpu.PrefetchScalarGridSpec(
        num_scalar_prefetch=1,
        grid=(GRID,),
        in_specs=[
            pl.BlockSpec((8, BLK_C), lambda i, idx: (idx[0] // 8, i)),
        ],
        out_specs=pl.BlockSpec((1, BLK_C), lambda i, idx: (0, i)),
    ),
    out_shape=jax.ShapeDtypeStruct((1, N_ROWS), jnp.float32),
    compiler_params=pltpu.CompilerParams(
        dimension_semantics=("arbitrary",)
    ),
)


def kernel(state, index):
    idx = jnp.asarray(index, jnp.int32).reshape(1)
    return _gather_row(idx, state.T).reshape(N_ROWS)


# speedup vs baseline: 2.2141x; 2.2141x over previous
import jax
import jax.numpy as jnp
from jax.experimental import pallas as pl
from jax.experimental.pallas import tpu as pltpu

N_ROWS = 16384
N_COLS = 16320


def _gather_row_body(idx_ref, block_ref, out_ref):
    s = idx_ref[0] % 8
    out_ref[...] = block_ref[pl.ds(s, 1), :]


_gather_row = pl.pallas_call(
    _gather_row_body,
    grid_spec=pltpu.PrefetchScalarGridSpec(
        num_scalar_prefetch=1,
        grid=(1,),
        in_specs=[
            pl.BlockSpec((8, N_ROWS), lambda i, idx: (idx[0] // 8, 0)),
        ],
        out_specs=pl.BlockSpec((1, N_ROWS), lambda i, idx: (0, 0)),
    ),
    out_shape=jax.ShapeDtypeStruct((1, N_ROWS), jnp.float32),
)


def kernel(state, index):
    idx = jnp.asarray(index, jnp.int32).reshape(1)
    return _gather_row(idx, state.T).reshape(N_ROWS)


# speedup vs baseline: 2.4852x; 1.1224x over previous
import jax
import jax.numpy as jnp
from jax.experimental import pallas as pl
from jax.experimental.pallas import tpu as pltpu

N_ROWS = 16384
N_COLS = 16320


def _gather_row_body(idx_ref, hbm_ref, out_ref, sem):
    row = idx_ref[0]
    pltpu.make_async_copy(
        hbm_ref.at[pl.ds(row, 1), :], out_ref, sem
    ).start()
    pltpu.make_async_copy(
        hbm_ref.at[pl.ds(row, 1), :], out_ref, sem
    ).wait()


_gather_row = pl.pallas_call(
    _gather_row_body,
    grid_spec=pltpu.PrefetchScalarGridSpec(
        num_scalar_prefetch=1,
        grid=(1,),
        in_specs=[pl.BlockSpec(memory_space=pl.ANY)],
        out_specs=pl.BlockSpec((1, N_ROWS), lambda i, idx: (0, 0)),
        scratch_shapes=[pltpu.SemaphoreType.DMA],
    ),
    out_shape=jax.ShapeDtypeStruct((1, N_ROWS), jnp.float32),
)


def kernel(state, index):
    idx = jnp.asarray(index, jnp.int32).reshape(1)
    return _gather_row(idx, state.T).reshape(N_ROWS)
